# baseline (speedup 1.0000x reference)
import functools

import jax
import jax.numpy as jnp
from jax import lax
from jax.experimental import pallas as pl
from jax.experimental.pallas import tpu as pltpu
from jax.experimental.pallas import tpu_sc as plsc

B, L, D = 4096, 200, 64
N = B * L


@functools.cache
def _build(d):
    info = plsc.get_sparse_core_info()
    nw = info.num_cores * info.num_subcores
    bg_n = B // 128
    lg_n = L // 8
    n_groups = L // 2
    mesh = plsc.VectorSubcoreMesh(core_axis_name="c", subcore_axis_name="s")

    @functools.partial(
        pl.kernel,
        mesh=mesh,
        out_type=jax.ShapeDtypeStruct((L, d // 8, bg_n, 8, 128), jnp.float32),
        scratch_types=[
            pltpu.VMEM((lg_n, 8, 128), jnp.int32),
            pltpu.VMEM((128, d), jnp.float32),
            pltpu.VMEM((128, d), jnp.float32),
            pltpu.VMEM((d // 8, 8, 128), jnp.float32),
            pltpu.VMEM((d // 8, 8, 128), jnp.float32),
            pltpu.SemaphoreType.DMA,
            pltpu.SemaphoreType.DMA,
            pltpu.SemaphoreType.DMA,
            pltpu.SemaphoreType.DMA,
        ],
        compiler_params=pltpu.CompilerParams(use_tc_tiling_on_sc=False,
                                             needs_layout_passes=False),
    )
    def gather_kernel(table_hbm, idx4_hbm, out_hbm, idx_all, rows0, rows1,
                      tbuf0, tbuf1, gsem0, gsem1, osem0, osem1):
        w = lax.axis_index("s") * info.num_cores + lax.axis_index("c")

        pltpu.sync_copy(idx4_hbm.at[:, w], idx_all)

        lane = lax.iota(jnp.int32, 16)
        row_ids = [lane + 16 * p for p in range(8)]

        def start_gather(l, rows, gsem):
            pltpu.async_copy(table_hbm.at[idx_all.at[l // 8, l % 8]], rows,
                             gsem)

        def transpose(rows, tbuf):
            for dg in range(d // 8):
                for di in range(8):
                    col = jnp.full((16,), dg * 8 + di, jnp.int32)
                    for p in range(8):
                        vec = plsc.load_gather(rows, [row_ids[p], col])
                        tbuf[dg, di, pl.ds(16 * p, 16)] = vec

        def start_writeback(l, tbuf, osem):
            pltpu.async_copy(tbuf, out_hbm.at[l, :, w], osem)

        def wait_gather(rows, sem):
            pltpu.make_async_copy(table_hbm.at[pl.ds(0, 128)], rows,
                                  sem).wait()

        def wait_writeback(tbuf, sem):
            pltpu.make_async_copy(tbuf, out_hbm.at[0, :, w], sem).wait()

        start_gather(0, rows0, gsem0)

        def body(g, carry):
            l0 = 2 * g

            wait_gather(rows0, gsem0)
            start_gather(l0 + 1, rows1, gsem1)

            @pl.when(g > 0)
            def _():
                wait_writeback(tbuf0, osem0)

            transpose(rows0, tbuf0)
            start_writeback(l0, tbuf0, osem0)

            wait_gather(rows1, gsem1)

            @pl.when(g < n_groups - 1)
            def _():
                start_gather(l0 + 2, rows0, gsem0)

            @pl.when(g > 0)
            def _():
                wait_writeback(tbuf1, osem1)

            transpose(rows1, tbuf1)
            start_writeback(l0 + 1, tbuf1, osem1)
            return carry

        lax.fori_loop(0, n_groups, body, 0)

        wait_writeback(tbuf0, osem0)
        wait_writeback(tbuf1, osem1)

    return gather_kernel


def kernel(batch, table):
    idx4 = batch.reshape(B // 128, 128, L // 8, 8).transpose(2, 0, 3, 1)
    out = _build(D)(table, idx4)
    return out.transpose(2, 4, 0, 1, 3).reshape(B, L, D)


# speedup vs baseline: 1.2538x; 1.2538x over previous
import functools

import jax
import jax.numpy as jnp
from jax import lax
from jax.experimental import pallas as pl
from jax.experimental.pallas import tpu as pltpu
from jax.experimental.pallas import tpu_sc as plsc

B, L, D = 4096, 200, 64
N = B * L


@functools.cache
def _build(d):
    info = plsc.get_sparse_core_info()
    nw = info.num_cores * info.num_subcores
    bg_n = B // 128
    lg_n = L // 8
    n_groups = L // 2
    mesh = plsc.VectorSubcoreMesh(core_axis_name="c", subcore_axis_name="s")

    @functools.partial(
        pl.kernel,
        mesh=mesh,
        out_type=jax.ShapeDtypeStruct((L, d // 8, bg_n, 8, 128), jnp.float32),
        scratch_types=[
            pltpu.VMEM((lg_n, 8, 128), jnp.int32),
            pltpu.VMEM((128, d), jnp.float32),
            pltpu.VMEM((128, d), jnp.float32),
            pltpu.VMEM((d // 8, 8, 128), jnp.float32),
            pltpu.VMEM((d // 8, 8, 128), jnp.float32),
            pltpu.SemaphoreType.DMA,
            pltpu.SemaphoreType.DMA,
            pltpu.SemaphoreType.DMA,
            pltpu.SemaphoreType.DMA,
        ],
        compiler_params=pltpu.CompilerParams(use_tc_tiling_on_sc=False,
                                             needs_layout_passes=False),
    )
    def gather_kernel(table_hbm, idx4_hbm, out_hbm, idx_all, rows0, rows1,
                      tbuf0, tbuf1, gsem0, gsem1, osem0, osem1):
        w = lax.axis_index("s") * info.num_cores + lax.axis_index("c")

        pltpu.sync_copy(idx4_hbm.at[:, w], idx_all)

        lane = lax.iota(jnp.int32, 16)
        row_ids = [lane + 16 * p for p in range(8)]

        def start_gather(l, rows, gsem):
            pltpu.async_copy(table_hbm.at[idx_all.at[l // 8, l % 8]], rows,
                             gsem)

        def transpose(rows, tbuf):
            def tr_body(dg, carry):
                col0 = jnp.full((16,), dg * 8, jnp.int32)
                for di in range(8):
                    col = col0 + di
                    vecs = [plsc.load_gather(rows, [row_ids[p], col])
                            for p in range(8)]
                    for p in range(8):
                        tbuf[dg, di, pl.ds(16 * p, 16)] = vecs[p]
                return carry

            lax.fori_loop(0, d // 8, tr_body, 0)

        def start_writeback(l, tbuf, osem):
            pltpu.async_copy(tbuf, out_hbm.at[l, :, w], osem)

        def wait_gather(rows, sem):
            pltpu.make_async_copy(table_hbm.at[pl.ds(0, 128)], rows,
                                  sem).wait()

        def wait_writeback(tbuf, sem):
            pltpu.make_async_copy(tbuf, out_hbm.at[0, :, w], sem).wait()

        start_gather(0, rows0, gsem0)

        def body(g, carry):
            l0 = 2 * g

            wait_gather(rows0, gsem0)
            start_gather(l0 + 1, rows1, gsem1)

            @pl.when(g > 0)
            def _():
                wait_writeback(tbuf0, osem0)

            transpose(rows0, tbuf0)
            start_writeback(l0, tbuf0, osem0)

            wait_gather(rows1, gsem1)

            @pl.when(g < n_groups - 1)
            def _():
                start_gather(l0 + 2, rows0, gsem0)

            @pl.when(g > 0)
            def _():
                wait_writeback(tbuf1, osem1)

            transpose(rows1, tbuf1)
            start_writeback(l0 + 1, tbuf1, osem1)
            return carry

        lax.fori_loop(0, n_groups, body, 0)

        wait_writeback(tbuf0, osem0)
        wait_writeback(tbuf1, osem1)

    return gather_kernel


def kernel(batch, table):
    idx4 = batch.reshape(B // 128, 128, L // 8, 8).transpose(2, 0, 3, 1)
    out = _build(D)(table, idx4)
    return out.transpose(2, 4, 0, 1, 3).reshape(B, L, D)


# speedup vs baseline: 2.0265x; 1.6163x over previous
import functools

import jax
import jax.numpy as jnp
from jax import lax
from jax.experimental import pallas as pl
from jax.experimental.pallas import tpu as pltpu
from jax.experimental.pallas import tpu_sc as plsc

B, L, D = 4096, 200, 64
N = B * L


@functools.cache
def _build(d):
    info = plsc.get_sparse_core_info()
    nw = info.num_cores * info.num_subcores
    bg_n = B // 128
    lg_n = L // 8
    n_groups = L // 2
    mesh = plsc.VectorSubcoreMesh(core_axis_name="c", subcore_axis_name="s")

    @functools.partial(
        pl.kernel,
        mesh=mesh,
        out_type=jax.ShapeDtypeStruct((L, d // 8, bg_n, 8, 128), jnp.float32),
        scratch_types=[
            pltpu.VMEM((lg_n, 8, 128), jnp.int32),
            pltpu.VMEM((128, d), jnp.float32),
            pltpu.VMEM((128, d), jnp.float32),
            pltpu.VMEM((d // 8, 8, 129), jnp.float32),
            pltpu.VMEM((d // 8, 8, 129), jnp.float32),
            pltpu.SemaphoreType.DMA,
            pltpu.SemaphoreType.DMA,
            pltpu.SemaphoreType.DMA,
            pltpu.SemaphoreType.DMA,
        ],
        compiler_params=pltpu.CompilerParams(use_tc_tiling_on_sc=False,
                                             needs_layout_passes=False),
    )
    def gather_kernel(table_hbm, idx4_hbm, out_hbm, idx_all, rows0, rows1,
                      tbuf0, tbuf1, gsem0, gsem1, osem0, osem1):
        w = lax.axis_index("s") * info.num_cores + lax.axis_index("c")

        pltpu.sync_copy(idx4_hbm.at[:, w], idx_all)

        lane = lax.iota(jnp.int32, 16)
        dg_ids = [(lane + 16 * q) // 8 for q in range(4)]
        di_ids = [(lane + 16 * q) % 8 for q in range(4)]

        def start_gather(l, rows, gsem):
            pltpu.async_copy(table_hbm.at[idx_all.at[l // 8, l % 8]], rows,
                             gsem)

        def transpose(rows, tbuf):
            def tr_body(bi, carry):
                bib = jnp.full((16,), bi, jnp.int32)
                for b2 in range(2):
                    for q in range(4):
                        vec = rows[2 * bi + b2, pl.ds(16 * q, 16)]
                        plsc.store_scatter(tbuf, [dg_ids[q], di_ids[q],
                                                  bib * 2 + b2], vec)
                return carry

            lax.fori_loop(0, 64, tr_body, 0)

        def start_writeback(l, tbuf, osem):
            pltpu.async_copy(tbuf.at[:, :, pl.ds(0, 128)],
                             out_hbm.at[l, :, w], osem)

        def wait_gather(rows, sem):
            pltpu.make_async_copy(table_hbm.at[pl.ds(0, 128)], rows,
                                  sem).wait()

        def wait_writeback(tbuf, sem):
            pltpu.make_async_copy(tbuf.at[:, :, pl.ds(0, 128)],
                                  out_hbm.at[0, :, w], sem).wait()

        start_gather(0, rows0, gsem0)

        def body(g, carry):
            l0 = 2 * g

            wait_gather(rows0, gsem0)
            start_gather(l0 + 1, rows1, gsem1)

            @pl.when(g > 0)
            def _():
                wait_writeback(tbuf0, osem0)

            transpose(rows0, tbuf0)
            start_writeback(l0, tbuf0, osem0)

            wait_gather(rows1, gsem1)

            @pl.when(g < n_groups - 1)
            def _():
                start_gather(l0 + 2, rows0, gsem0)

            @pl.when(g > 0)
            def _():
                wait_writeback(tbuf1, osem1)

            transpose(rows1, tbuf1)
            start_writeback(l0 + 1, tbuf1, osem1)
            return carry

        lax.fori_loop(0, n_groups, body, 0)

        wait_writeback(tbuf0, osem0)
        wait_writeback(tbuf1, osem1)

    return gather_kernel


def kernel(batch, table):
    idx4 = batch.reshape(B // 128, 128, L // 8, 8).transpose(2, 0, 3, 1)
    out = _build(D)(table, idx4)
    return out.transpose(2, 4, 0, 1, 3).reshape(B, L, D)
